# Initial kernel scaffold; baseline (speedup 1.0000x reference)
import jax
import jax.numpy as jnp
from jax.experimental import pallas as pl


def kernel(inputs, targets):
    raise NotImplementedError("write your pallas kernel here")


# speedup vs baseline: 22.4315x; 22.4315x over previous
import functools

import jax
import jax.numpy as jnp
from jax import lax
from jax.experimental import pallas as pl
from jax.experimental.pallas import tpu as pltpu

N_TOK = 32768
N_CLS = 128
BLK = 1024
N_BLKS = N_TOK // BLK


def _stage1_body(x_ref, t_ref, ce_ref, acc_ref):
    i = pl.program_id(0)
    x = x_ref[0]
    t = t_ref[0]
    m = jnp.max(x, axis=1, keepdims=True)
    e = jnp.exp(x - m)
    s = jnp.sum(e, axis=1, keepdims=True)
    lse = jnp.log(s) + m
    x0 = x[:, 0:1]
    x1 = x[:, 1:2]
    gold = x0 + t * (x1 - x0)
    ce = lse - gold
    is_pos = t > 0.5
    pt = jnp.exp(-ce)
    focal = (1.0 - pt) * (1.0 - pt) * ce
    n_pos_blk = jnp.sum(t)
    s_pos_blk = jnp.sum(jnp.where(is_pos, focal, 0.0))
    s_negall_blk = jnp.sum(jnp.where(is_pos, 0.0, 0.25 * focal))
    ce_ref[0] = jnp.where(is_pos, -1.0, ce)

    lanes = lax.broadcasted_iota(jnp.int32, (1, 128), 1)
    upd = (jnp.where(lanes == 0, n_pos_blk, 0.0)
           + jnp.where(lanes == 1, s_pos_blk, 0.0)
           + jnp.where(lanes == 2, s_negall_blk, 0.0))

    @pl.when(i == 0)
    def _():
        acc_ref[...] = jnp.zeros_like(acc_ref)

    acc_ref[...] += upd


def _stage2_body(scal_ref, nc_ref, out_ref):
    v = nc_ref[...]
    kbits = lax.bitcast_convert_type(v, jnp.int32)
    key = jnp.maximum(kbits, 0)

    n_pos_f = scal_ref[0]
    s_pos = scal_ref[1]
    s_negall = scal_ref[2]
    n_pos = n_pos_f.astype(jnp.int32)
    n_neg = N_TOK - n_pos
    k = jnp.minimum(n_pos // 4, n_neg)

    def bit_step(j, prefix):
        b = 30 - j
        cand = prefix | (1 << b)
        cnt = jnp.sum((key >= cand).astype(jnp.int32))
        return jnp.where(cnt >= k, cand, prefix)

    tau_bits = lax.fori_loop(0, 31, bit_step, jnp.int32(0))
    tau = lax.bitcast_convert_type(tau_bits, jnp.float32)

    gt = key > tau_bits
    cnt_gt = jnp.sum(gt.astype(jnp.int32))
    fl = 0.25 * (1.0 - jnp.exp(-v)) * (1.0 - jnp.exp(-v)) * v
    s_gt = jnp.sum(jnp.where(gt, fl, 0.0))
    g_tau = 0.25 * (1.0 - jnp.exp(-tau)) * (1.0 - jnp.exp(-tau)) * tau
    s_hard = jnp.where(k > 0,
                       s_gt + (k - cnt_gt).astype(jnp.float32) * g_tau,
                       0.0)
    n_sel = n_pos + k
    sel_mean = (s_pos + s_hard) / jnp.maximum(n_sel, 1).astype(jnp.float32)
    full_mean = (s_pos + s_negall) / jnp.float32(N_TOK)
    out_ref[...] = jnp.where(n_pos == 0, full_mean, sel_mean)[None, None]


@jax.jit
def kernel(inputs, targets):
    x3 = inputs.reshape(N_BLKS, BLK, N_CLS)
    t3 = targets.reshape(N_BLKS, BLK, 1).astype(jnp.float32)

    neg_ce, acc = pl.pallas_call(
        _stage1_body,
        grid=(N_BLKS,),
        in_specs=[
            pl.BlockSpec((1, BLK, N_CLS), lambda i: (i, 0, 0)),
            pl.BlockSpec((1, BLK, 1), lambda i: (i, 0, 0)),
        ],
        out_specs=[
            pl.BlockSpec((1, BLK, 1), lambda i: (i, 0, 0)),
            pl.BlockSpec((1, 128), lambda i: (0, 0)),
        ],
        out_shape=[
            jax.ShapeDtypeStruct((N_BLKS, BLK, 1), jnp.float32),
            jax.ShapeDtypeStruct((1, 128), jnp.float32),
        ],
    )(x3, t3)

    scal = acc.reshape(128)[:8]
    nc2 = neg_ce.reshape(N_TOK // 128, 128)

    out = pl.pallas_call(
        _stage2_body,
        in_specs=[
            pl.BlockSpec(memory_space=pltpu.SMEM),
            pl.BlockSpec((N_TOK // 128, 128), lambda: (0, 0)),
        ],
        out_specs=pl.BlockSpec((1, 1), lambda: (0, 0)),
        out_shape=jax.ShapeDtypeStruct((1, 1), jnp.float32),
    )(scal, nc2)

    return out[0, 0]


# speedup vs baseline: 42.7433x; 1.9055x over previous
import functools

import jax
import jax.numpy as jnp
from jax import lax
from jax.experimental import pallas as pl
from jax.experimental.pallas import tpu as pltpu

N_TOK = 32768
N_CLS = 128
BLK = 1024
N_BLKS = N_TOK // BLK


def _stage1_body(x_ref, t_ref, ce_ref, acc_ref):
    i = pl.program_id(0)
    x = x_ref[0]
    t = t_ref[0]
    m = jnp.max(x, axis=2)
    e = jnp.exp(x - m[:, :, None])
    s = jnp.sum(e, axis=2)
    lse = jnp.log(s) + m
    cls = lax.broadcasted_iota(jnp.int32, (8, 128, N_CLS), 2)
    gold = jnp.sum(jnp.where(cls == t[:, :, None], x, 0.0), axis=2)
    ce = lse - gold
    is_pos = t > 0
    tf = t.astype(jnp.float32)
    pt = jnp.exp(-ce)
    focal = (1.0 - pt) * (1.0 - pt) * ce
    n_pos_blk = jnp.sum(tf)
    s_pos_blk = jnp.sum(jnp.where(is_pos, focal, 0.0))
    s_negall_blk = jnp.sum(jnp.where(is_pos, 0.0, 0.25 * focal))
    ce_ref[0] = jnp.where(is_pos, -1.0, ce)

    lanes = lax.broadcasted_iota(jnp.int32, (1, 128), 1)
    upd = (jnp.where(lanes == 0, n_pos_blk, 0.0)
           + jnp.where(lanes == 1, s_pos_blk, 0.0)
           + jnp.where(lanes == 2, s_negall_blk, 0.0))

    @pl.when(i == 0)
    def _():
        acc_ref[...] = jnp.zeros_like(acc_ref)

    acc_ref[...] += upd


def _stage2_body(scal_ref, nc_ref, out_ref):
    v = nc_ref[...]
    kbits = lax.bitcast_convert_type(v, jnp.int32)
    key = jnp.maximum(kbits, 0)

    n_pos_f = scal_ref[0]
    s_pos = scal_ref[1]
    s_negall = scal_ref[2]
    n_pos = n_pos_f.astype(jnp.int32)
    n_neg = N_TOK - n_pos
    k = jnp.minimum(n_pos // 4, n_neg)

    def bit_step(j, prefix):
        b = 30 - j
        cand = prefix | (1 << b)
        cnt = jnp.sum((key >= cand).astype(jnp.int32))
        return jnp.where(cnt >= k, cand, prefix)

    tau_bits = lax.fori_loop(0, 31, bit_step, jnp.int32(0))
    tau = lax.bitcast_convert_type(tau_bits, jnp.float32)

    gt = key > tau_bits
    cnt_gt = jnp.sum(gt.astype(jnp.int32))
    fl = 0.25 * (1.0 - jnp.exp(-v)) * (1.0 - jnp.exp(-v)) * v
    s_gt = jnp.sum(jnp.where(gt, fl, 0.0))
    g_tau = 0.25 * (1.0 - jnp.exp(-tau)) * (1.0 - jnp.exp(-tau)) * tau
    s_hard = jnp.where(k > 0,
                       s_gt + (k - cnt_gt).astype(jnp.float32) * g_tau,
                       0.0)
    n_sel = n_pos + k
    sel_mean = (s_pos + s_hard) / jnp.maximum(n_sel, 1).astype(jnp.float32)
    full_mean = (s_pos + s_negall) / jnp.float32(N_TOK)
    out_ref[...] = jnp.where(n_pos == 0, full_mean, sel_mean)[None, None]


@jax.jit
def kernel(inputs, targets):
    x4 = inputs.reshape(N_BLKS, 8, 128, N_CLS)
    t3 = targets.reshape(N_BLKS, 8, 128)

    neg_ce, acc = pl.pallas_call(
        _stage1_body,
        grid=(N_BLKS,),
        in_specs=[
            pl.BlockSpec((1, 8, 128, N_CLS), lambda i: (i, 0, 0, 0)),
            pl.BlockSpec((1, 8, 128), lambda i: (i, 0, 0)),
        ],
        out_specs=[
            pl.BlockSpec((1, 8, 128), lambda i: (i, 0, 0)),
            pl.BlockSpec((1, 128), lambda i: (0, 0)),
        ],
        out_shape=[
            jax.ShapeDtypeStruct((N_BLKS, 8, 128), jnp.float32),
            jax.ShapeDtypeStruct((1, 128), jnp.float32),
        ],
    )(x4, t3)

    scal = acc.reshape(128)[:8]
    nc2 = neg_ce.reshape(N_TOK // 128, 128)

    out = pl.pallas_call(
        _stage2_body,
        in_specs=[
            pl.BlockSpec(memory_space=pltpu.SMEM),
            pl.BlockSpec((N_TOK // 128, 128), lambda: (0, 0)),
        ],
        out_specs=pl.BlockSpec((1, 1), lambda: (0, 0)),
        out_shape=jax.ShapeDtypeStruct((1, 1), jnp.float32),
    )(scal, nc2)

    return out[0, 0]
